# baseline (speedup 1.0000x reference)
import functools

import jax
import jax.numpy as jnp
from jax import lax
from jax.experimental import pallas as pl
from jax.experimental.pallas import tpu as pltpu
from jax.experimental.pallas import tpu_sc as plsc

N_NODES = 10000
N_PAD = 10240
EMB = 128
N_EDGES = 320000
NC = 2
NS = 16
HALF = N_PAD // NC
TRASH = 128
ACC_ROWS = HALF + TRASH
EDGES_PER_TILE = N_EDGES // NS
CHUNK = 128
N_PHASES = 2
PH_CHUNKS = 80
N_CHUNKS = N_PHASES * PH_CHUNKS
EDGES_PER_TILE_PAD = N_CHUNKS * CHUNK


def _spmm_sc(x, src_t, dst_t, w_t):
  mesh = plsc.VectorSubcoreMesh(core_axis_name="c", subcore_axis_name="s")

  @functools.partial(
      pl.kernel,
      mesh=mesh,
      out_type=jax.ShapeDtypeStruct((N_PAD, EMB), jnp.float32),
      scratch_types=[
          pltpu.VMEM((PH_CHUNKS, CHUNK), jnp.int32),
          pltpu.VMEM((PH_CHUNKS, CHUNK), jnp.int32),
          pltpu.VMEM((PH_CHUNKS, CHUNK), jnp.float32),
          pltpu.VMEM((2, CHUNK, EMB), jnp.float32),
          pltpu.VMEM_SHARED((ACC_ROWS, EMB), jnp.float32),
          pltpu.SemaphoreType.DMA,
          pltpu.SemaphoreType.DMA,
          pltpu.SemaphoreType.DMA,
          pltpu.SemaphoreType.DMA,
      ],
  )
  def spmm(x_hbm, src_hbm, dst_hbm, w_hbm, out_hbm,
           src_v, dst_v, w_v, rows_v, acc_sh, sg0, sg1, ss0, ss1):
    sem_g = (sg0, sg1)
    sem_s = (ss0, ss1)
    c = lax.axis_index("c")
    s = lax.axis_index("s")
    lo = c * HALF

    z16 = jnp.zeros((16,), jnp.float32)

    def zfill(i, _):
      r = i // (EMB // 16)
      j = i % (EMB // 16)
      rows_v[0, r, pl.ds(j * 16, 16)] = z16
      return 0

    lax.fori_loop(0, CHUNK * (EMB // 16), zfill, 0)
    rows_per_tile = ACC_ROWS // NS
    base = s * rows_per_tile
    pltpu.sync_copy(rows_v.at[0], acc_sh.at[pl.ds(base, CHUNK)])
    pltpu.sync_copy(rows_v.at[0], acc_sh.at[pl.ds(base + CHUNK, CHUNK)])
    pltpu.sync_copy(rows_v.at[0, pl.ds(0, rows_per_tile - 2 * CHUNK)],
                    acc_sh.at[pl.ds(base + 2 * CHUNK,
                                    rows_per_tile - 2 * CHUNK)])
    plsc.subcore_barrier()

    def scale(k, b):
      def group_body(g, _):
        wg = w_v[k, pl.ds(g * 16, 16)]
        for e in range(16):
          we = wg[e]
          row = g * 16 + e
          for j in range(EMB // 16):
            sl = pl.ds(j * 16, 16)
            rows_v[b, row, sl] = rows_v[b, row, sl] * we
        return 0

      lax.fori_loop(0, CHUNK // 16, group_body, 0)

    for ph in range(N_PHASES):
      pltpu.sync_copy(src_hbm.at[s, ph], src_v)
      pltpu.sync_copy(dst_hbm.at[s, ph], dst_v)
      pltpu.sync_copy(w_hbm.at[s, ph], w_v)

      def remap_body(i, _):
        kk = i // (CHUNK // 16)
        g = i % (CHUNK // 16)
        sl = pl.ds(g * 16, 16)
        d = dst_v[kk, sl] - lo
        valid = (d >= 0) & (d < HALF)
        dst_v[kk, sl] = jnp.where(valid, d, HALF)
        return 0

      lax.fori_loop(0, PH_CHUNKS * (CHUNK // 16), remap_body, 0)

      def quad_body(q, _):
        k = q * 4
        g0 = pltpu.async_copy(x_hbm.at[src_v.at[k]], rows_v.at[0], sg0)
        g1 = pltpu.async_copy(x_hbm.at[src_v.at[k + 1]], rows_v.at[1], sg1)
        g0.wait()
        scale(k, 0)
        s0 = pltpu.async_copy(rows_v.at[0], acc_sh.at[dst_v.at[k]], ss0,
                              add=True)
        g1.wait()
        scale(k + 1, 1)
        s0.wait()
        g2 = pltpu.async_copy(x_hbm.at[src_v.at[k + 2]], rows_v.at[0], sg0)
        s1 = pltpu.async_copy(rows_v.at[1], acc_sh.at[dst_v.at[k + 1]], ss1,
                              add=True)
        g2.wait()
        scale(k + 2, 0)
        s1.wait()
        g3 = pltpu.async_copy(x_hbm.at[src_v.at[k + 3]], rows_v.at[1], sg1)
        s2 = pltpu.async_copy(rows_v.at[0], acc_sh.at[dst_v.at[k + 2]], ss0,
                              add=True)
        g3.wait()
        scale(k + 3, 1)
        s2.wait()
        s3 = pltpu.async_copy(rows_v.at[1], acc_sh.at[dst_v.at[k + 3]], ss1,
                              add=True)
        s3.wait()
        return 0

      lax.fori_loop(0, PH_CHUNKS // 4, quad_body, 0)

    plsc.subcore_barrier()

    out_rows = HALF // NS
    pltpu.sync_copy(acc_sh.at[pl.ds(s * out_rows, out_rows)],
                    out_hbm.at[pl.ds(c * HALF + s * out_rows, out_rows)])

  return spmm(x, src_t, dst_t, w_t)


_ROWS_BLK = 1024


def _finalize(bvec, x0, x1, x2):
  def body(b_ref, x0_ref, x1_ref, x2_ref, o_ref):
    def n(v):
      ss = jnp.sum(v * v, axis=-1, keepdims=True)
      nrm = jnp.sqrt(ss)
      return v / jnp.maximum(nrm, 1e-12)

    acc = (b_ref[0] * n(x0_ref[...]) + b_ref[1] * n(x1_ref[...])
           + b_ref[2] * n(x2_ref[...]))
    o_ref[...] = 2.0 * acc

  blk = lambda: pl.BlockSpec((_ROWS_BLK, EMB), lambda i: (i, 0))
  return pl.pallas_call(
      body,
      grid=(N_PAD // _ROWS_BLK,),
      in_specs=[pl.BlockSpec(memory_space=pltpu.SMEM), blk(), blk(), blk()],
      out_specs=blk(),
      out_shape=jax.ShapeDtypeStruct((N_PAD, EMB), jnp.float32),
  )(bvec, x0, x1, x2)


def kernel(nodes_emb, edge_weight, b, edge_index):
  def tile_pad(a, fill):
    a = a.reshape(NS, EDGES_PER_TILE)
    a = jnp.pad(a, ((0, 0), (0, EDGES_PER_TILE_PAD - EDGES_PER_TILE)),
                constant_values=fill)
    return a.reshape(NS, N_PHASES, PH_CHUNKS, CHUNK)

  src_t = tile_pad(edge_index[0], 0)
  dst_t = tile_pad(edge_index[1], N_PAD)
  w_t = tile_pad(edge_weight, 0.0)
  bvec = b.reshape(3)

  x0 = jnp.pad(nodes_emb, ((0, N_PAD - N_NODES), (0, 0)))
  x1 = _spmm_sc(x0, src_t, dst_t, w_t)
  x2 = _spmm_sc(x1, src_t, dst_t, w_t)
  core = _finalize(bvec, x0, x1, x2)
  zeros = jnp.zeros((1, EMB), jnp.float32)
  return jnp.concatenate([zeros, core[:N_NODES]], axis=0)


# speedup vs baseline: 1.7698x; 1.7698x over previous
import functools

import jax
import jax.numpy as jnp
from jax import lax
from jax.experimental import pallas as pl
from jax.experimental.pallas import tpu as pltpu
from jax.experimental.pallas import tpu_sc as plsc

N_NODES = 10000
N_PAD = 10240
EMB = 128
N_EDGES = 320000
NC = 2
NS = 16
N_TILES = NC * NS
EDGES_PER_TILE = N_EDGES // N_TILES
CHUNK = 128
N_CHUNKS = -(-EDGES_PER_TILE // CHUNK) + 1
EDGES_PER_TILE_PAD = N_CHUNKS * CHUNK


def _spmm_sc(x, src_t, dst_t, w_t):
  mesh = plsc.VectorSubcoreMesh(core_axis_name="c", subcore_axis_name="s")

  @functools.partial(
      pl.kernel,
      mesh=mesh,
      out_type=jax.ShapeDtypeStruct((NC, N_PAD, EMB), jnp.float32),
      scratch_types=[
          pltpu.VMEM((N_CHUNKS, CHUNK), jnp.int32),
          pltpu.VMEM((N_CHUNKS, CHUNK), jnp.int32),
          pltpu.VMEM((N_CHUNKS, CHUNK), jnp.float32),
          pltpu.VMEM((CHUNK, EMB), jnp.float32),
          pltpu.VMEM_SHARED((N_PAD, EMB), jnp.float32),
          pltpu.SemaphoreType.DMA,
      ],
  )
  def spmm(x_hbm, src_hbm, dst_hbm, w_hbm, out_hbm,
           src_v, dst_v, w_v, rows_v, acc_sh, sem):
    c = lax.axis_index("c")
    s = lax.axis_index("s")

    z16 = jnp.zeros((16,), jnp.float32)

    def zfill(i, _):
      r = i // (EMB // 16)
      j = i % (EMB // 16)
      rows_v[r, pl.ds(j * 16, 16)] = z16
      return 0

    lax.fori_loop(0, CHUNK * (EMB // 16), zfill, 0)
    rows_per_tile = N_PAD // NS
    base = s * rows_per_tile
    for q in range(rows_per_tile // CHUNK):
      pltpu.sync_copy(rows_v, acc_sh.at[pl.ds(base + q * CHUNK, CHUNK)])

    pltpu.sync_copy(src_hbm.at[c, s], src_v)
    pltpu.sync_copy(dst_hbm.at[c, s], dst_v)
    pltpu.sync_copy(w_hbm.at[c, s], w_v)
    plsc.subcore_barrier()

    def chunk_body(k, _):
      pltpu.async_copy(x_hbm.at[src_v.at[k]], rows_v, sem).wait()

      def group_body(g, _):
        wg = w_v[k, pl.ds(g * 16, 16)]
        for e in range(16):
          we = wg[e]
          row = g * 16 + e
          for j in range(EMB // 16):
            sl = pl.ds(j * 16, 16)
            rows_v[row, sl] = rows_v[row, sl] * we
        return 0

      lax.fori_loop(0, CHUNK // 16, group_body, 0)

      pltpu.sync_copy(rows_v, acc_sh.at[dst_v.at[k]], add=True)
      return 0

    lax.fori_loop(0, N_CHUNKS, chunk_body, 0)
    plsc.subcore_barrier()

    sl = pl.ds(s * rows_per_tile, rows_per_tile)
    pltpu.sync_copy(acc_sh.at[sl], out_hbm.at[c, sl])

  return spmm(x, src_t, dst_t, w_t)


_ROWS_BLK = 1024


def _combine(p):
  def body(p_ref, o_ref):
    o_ref[...] = p_ref[0] + p_ref[1]

  return pl.pallas_call(
      body,
      grid=(N_PAD // _ROWS_BLK,),
      in_specs=[pl.BlockSpec((2, _ROWS_BLK, EMB), lambda i: (0, i, 0))],
      out_specs=pl.BlockSpec((_ROWS_BLK, EMB), lambda i: (i, 0)),
      out_shape=jax.ShapeDtypeStruct((N_PAD, EMB), jnp.float32),
  )(p)


def _finalize(bvec, x0, x1, p2):
  def body(b_ref, x0_ref, x1_ref, p2_ref, o_ref):
    x2 = p2_ref[0] + p2_ref[1]

    def n(v):
      ss = jnp.sum(v * v, axis=-1, keepdims=True)
      nrm = jnp.sqrt(ss)
      return v / jnp.maximum(nrm, 1e-12)

    acc = (b_ref[0] * n(x0_ref[...]) + b_ref[1] * n(x1_ref[...])
           + b_ref[2] * n(x2))
    o_ref[...] = 2.0 * acc

  blk = lambda: pl.BlockSpec((_ROWS_BLK, EMB), lambda i: (i, 0))
  return pl.pallas_call(
      body,
      grid=(N_PAD // _ROWS_BLK,),
      in_specs=[
          pl.BlockSpec(memory_space=pltpu.SMEM),
          blk(),
          blk(),
          pl.BlockSpec((2, _ROWS_BLK, EMB), lambda i: (0, i, 0)),
      ],
      out_specs=blk(),
      out_shape=jax.ShapeDtypeStruct((N_PAD, EMB), jnp.float32),
  )(bvec, x0, x1, p2)


def kernel(nodes_emb, edge_weight, b, edge_index):
  def tile_pad(a, fill):
    a = a.reshape(N_TILES, EDGES_PER_TILE)
    a = jnp.pad(a, ((0, 0), (0, EDGES_PER_TILE_PAD - EDGES_PER_TILE)),
                constant_values=fill)
    return a.reshape(NC, NS, N_CHUNKS, CHUNK)

  src_t = tile_pad(edge_index[0], 0)
  dst_t = tile_pad(edge_index[1], 0)
  w_t = tile_pad(edge_weight, 0.0)
  bvec = b.reshape(3)

  x0 = jnp.pad(nodes_emb, ((0, N_PAD - N_NODES), (0, 0)))
  p1 = _spmm_sc(x0, src_t, dst_t, w_t)
  x1 = _combine(p1)
  p2 = _spmm_sc(x1, src_t, dst_t, w_t)
  core = _finalize(bvec, x0, x1, p2)
  zeros = jnp.zeros((1, EMB), jnp.float32)
  return jnp.concatenate([zeros, core[:N_NODES]], axis=0)
